# baseline (speedup 1.0000x reference)
import functools

import jax
import jax.numpy as jnp
from jax import lax
from jax.experimental import pallas as pl
from jax.experimental.pallas import tpu as pltpu
from jax.experimental.pallas import tpu_sc as plsc

PADDING_IDX = 1
LANES = 16
CHUNK = 32


def _make_sc_kernel(bsz, seq_len, n_table, dim, n_workers, n_cores):
    s_per_w = seq_len // n_workers
    n_chunks_per_b = s_per_w // CHUNK
    mesh = plsc.VectorSubcoreMesh(core_axis_name="c", subcore_axis_name="s")

    @functools.partial(
        pl.kernel,
        mesh=mesh,
        out_type=jax.ShapeDtypeStruct((bsz, seq_len, dim), jnp.float32),
        scratch_types=[
            pltpu.VMEM((bsz, s_per_w), jnp.int32),
            pltpu.VMEM((bsz, s_per_w), jnp.int32),
            pltpu.VMEM((CHUNK, dim), jnp.float32),
            pltpu.VMEM((CHUNK, dim), jnp.float32),
            pltpu.SemaphoreType.DMA,
            pltpu.SemaphoreType.DMA,
            pltpu.SemaphoreType.DMA,
            pltpu.SemaphoreType.DMA,
        ],
    )
    def sc_posemb(inp_hbm, w_hbm, out_hbm, inp_v, pos_v, b0, b1, g0, g1, w0, w1):
        wid = lax.axis_index("s") * n_cores + lax.axis_index("c")
        base_s = wid * s_per_w
        for b in range(bsz):
            pltpu.sync_copy(inp_hbm.at[b, pl.ds(base_s, s_per_w)], inp_v.at[b])
        zero = jnp.zeros((LANES,), jnp.int32)
        one = jnp.full((LANES,), 1, jnp.int32)
        pad = jnp.full((LANES,), PADDING_IDX, jnp.int32)
        for j in range(s_per_w // LANES):
            sl = pl.ds(j * LANES, LANES)
            cum = zero
            for b in range(bsz):
                m = jnp.where(inp_v[b, sl] == pad, zero, one)
                cum = cum + m
                pos_v[b, sl] = cum * m + pad

        chunks = [(b, c) for b in range(bsz) for c in range(n_chunks_per_b)]
        bufs, gsem, wsem = [b0, b1], [g0, g1], [w0, w1]
        gh = [None, None]
        wh = [None, None]

        def start_gather(k, i):
            b, c = chunks[k]
            idx = pos_v.at[b, pl.ds(c * CHUNK, CHUNK)]
            gh[i] = pltpu.async_copy(w_hbm.at[idx], bufs[i], gsem[i])

        def start_write(k, i):
            b, c = chunks[k]
            dst = out_hbm.at[b, pl.ds(base_s + c * CHUNK, CHUNK)]
            wh[i] = pltpu.async_copy(bufs[i], dst, wsem[i])

        start_gather(0, 0)
        for k in range(len(chunks)):
            i = k % 2
            gh[i].wait()
            if k + 1 < len(chunks):
                o = (k + 1) % 2
                if wh[o] is not None:
                    wh[o].wait()
                start_gather(k + 1, o)
            start_write(k, i)
        wh[0].wait()
        wh[1].wait()

    return sc_posemb


def kernel(input, weights):
    bsz, seq_len = input.shape
    n_table, dim = weights.shape
    info = plsc.get_sparse_core_info()
    n_workers = info.num_cores * info.num_subcores
    sc = _make_sc_kernel(bsz, seq_len, n_table, dim, n_workers, info.num_cores)
    return sc(input, weights)


# speedup vs baseline: 16.9168x; 16.9168x over previous
import jax
import jax.numpy as jnp
from jax.experimental import pallas as pl

PADDING_IDX = 1
SEQ_BLOCK = 512


def _posemb_block(inp_ref, w_ref, out_ref):
    inp = inp_ref[...].T
    s, bsz = inp.shape
    mask = (inp != PADDING_IDX).astype(jnp.int32)
    cols = []
    acc = jnp.zeros_like(mask[:, 0:1])
    for b in range(bsz):
        acc = acc + mask[:, b : b + 1]
        cols.append(acc)
    pos = jnp.concatenate(cols, axis=1) * mask + PADDING_IDX
    b_id = pl.program_id(0)
    zero = jnp.zeros_like(pos[:, 0:1])
    pos_b = zero
    for b in range(bsz):
        pos_b = pos_b + jnp.where(b_id == b, pos[:, b : b + 1], zero)
    n_rows = w_ref.shape[0]
    iota = jax.lax.broadcasted_iota(jnp.int32, (1, n_rows), 1)
    onehot = (pos_b == iota).astype(jnp.float32)
    out_ref[...] = jnp.dot(onehot, w_ref[...], preferred_element_type=jnp.float32)


def kernel(input, weights):
    bsz, seq_len = input.shape
    dim = weights.shape[1]
    n_rows = max(8, bsz + 2)
    n_j = seq_len // SEQ_BLOCK
    out = pl.pallas_call(
        _posemb_block,
        grid=(bsz, n_j),
        in_specs=[
            pl.BlockSpec((bsz, SEQ_BLOCK), lambda b, j: (0, j)),
            pl.BlockSpec((n_rows, dim), lambda b, j: (0, 0)),
        ],
        out_specs=pl.BlockSpec(
            (SEQ_BLOCK, dim), lambda b, j: (b * n_j + j, 0)
        ),
        out_shape=jax.ShapeDtypeStruct((bsz * seq_len, dim), weights.dtype),
    )(input, weights)
    return out.reshape(bsz, seq_len, dim)
